# baseline (speedup 1.0000x reference)
import functools

import jax
import jax.numpy as jnp
from jax.experimental import pallas as pl
from jax.experimental.pallas import tpu as pltpu


def _dot1(a, b):
    return jnp.dot(a.astype(jnp.bfloat16), b.astype(jnp.bfloat16),
                   preferred_element_type=jnp.float32)


def _dot1t(a, w):
    return jax.lax.dot_general(
        a.astype(jnp.bfloat16), w.astype(jnp.bfloat16),
        (((1,), (1,)), ((), ())), preferred_element_type=jnp.float32)


def _head1_block(h, i, m, s_ref, c_ref, ssc_ref):
    @pl.when(m == 0)
    def _():
        c_ref[i] = jnp.mean(h, axis=0, keepdims=True)
        s_ref[i] = jnp.zeros_like(s_ref[i])
        ssc_ref[i] = jnp.zeros_like(ssc_ref[i])

    d = h - c_ref[i]
    s_ref[i] += jnp.sum(h, axis=0, keepdims=True)
    ssc_ref[i] += jnp.sum(d * d, axis=0, keepdims=True)


def _gcn1_kernel(x_in_ref, w_ref, adj1_ref, adj2_ref, b_ref, a_ref,
                 xe_ref, xn_ref,
                 hwe_ref, hwn_ref, hbe_ref, hbn_ref,
                 o_ref, he_ref, hn_ref, s_ref, c_ref, ssc_ref, xv_ref, *, bm2):
    m = pl.program_id(0)

    @pl.when(m == 0)
    def _():
        xv_ref[...] = _dot1t(x_in_ref[...], w_ref[...])

    o_ref[...] = jnp.zeros_like(o_ref)
    o_ref[pl.ds(0, bm2), :] += _dot1(adj1_ref[...], xv_ref[...])
    o_ref[pl.ds(bm2, bm2), :] += _dot1(adj2_ref[...], xv_ref[...])
    h = o_ref[...] + b_ref[...]
    a = a_ref[0]
    o_ref[...] = jnp.where(h >= 0, h, a * h)

    for i, (x_ref, h_ref, hw_ref, hb_ref) in enumerate(
            ((xe_ref, he_ref, hwe_ref, hbe_ref),
             (xn_ref, hn_ref, hwn_ref, hbn_ref))):
        h1 = _dot1t(x_ref[...], hw_ref[...]) + hb_ref[...]
        h_ref[...] = h1
        _head1_block(h1, i, m, s_ref, c_ref, ssc_ref)


def _gcn1_layer(x_in, w, adj, b, alpha, ego, nbr, hwe, hwn, hbe, hbn, bm):
    n, k = adj.shape
    dout = w.shape[0]
    hh = hwe.shape[0]
    row = lambda m: (m, 0)
    const2 = lambda m: (0, 0)
    const3 = lambda m: (0, 0, 0)
    bm2 = bm // 2
    return pl.pallas_call(
        functools.partial(_gcn1_kernel, bm2=bm2),
        grid=(n // bm,),
        in_specs=[
            pl.BlockSpec((n, x_in.shape[1]), const2),
            pl.BlockSpec((dout, x_in.shape[1]), const2),
            pl.BlockSpec((bm2, k), lambda m: (2 * m, 0)),
            pl.BlockSpec((bm2, k), lambda m: (2 * m + 1, 0)),
            pl.BlockSpec((1, dout), const2),
            pl.BlockSpec(memory_space=pltpu.SMEM),
            pl.BlockSpec((bm, ego.shape[1]), row),
            pl.BlockSpec((bm, nbr.shape[1]), row),
            pl.BlockSpec((hh, ego.shape[1]), const2),
            pl.BlockSpec((hh, nbr.shape[1]), const2),
            pl.BlockSpec((1, hh), const2),
            pl.BlockSpec((1, hh), const2),
        ],
        out_specs=[
            pl.BlockSpec((bm, dout), row),
            pl.BlockSpec((bm, hh), row),
            pl.BlockSpec((bm, hh), row),
            pl.BlockSpec((2, 1, hh), const3),
            pl.BlockSpec((2, 1, hh), const3),
            pl.BlockSpec((2, 1, hh), const3),
        ],
        out_shape=[
            jax.ShapeDtypeStruct((n, dout), jnp.float32),
            jax.ShapeDtypeStruct((n, hh), jnp.float32),
            jax.ShapeDtypeStruct((n, hh), jnp.float32),
            jax.ShapeDtypeStruct((2, 1, hh), jnp.float32),
            jax.ShapeDtypeStruct((2, 1, hh), jnp.float32),
            jax.ShapeDtypeStruct((2, 1, hh), jnp.float32),
        ],
        scratch_shapes=[pltpu.VMEM((n, dout), jnp.float32)],
        compiler_params=pltpu.CompilerParams(
            dimension_semantics=("arbitrary",),
        ),
    )(x_in, w, adj, adj, b.reshape(1, dout), alpha.reshape(1), ego, nbr,
      hwe, hwn, hbe.reshape(1, hh), hbn.reshape(1, hh))


def _gcn2_kernel(x_in_ref, w_ref, adj1_ref, adj2_ref, b_ref, a_ref,
                 hw_ref, hb_ref,
                 o_ref, hz_ref, s_ref, c_ref, ssc_ref, xv_ref, *, bm2):
    m = pl.program_id(0)

    @pl.when(m == 0)
    def _():
        xv_ref[...] = _dot1t(x_in_ref[...], w_ref[...])

    o_ref[...] = jnp.zeros_like(o_ref)
    o_ref[pl.ds(0, bm2), :] += _dot1(adj1_ref[...], xv_ref[...])
    o_ref[pl.ds(bm2, bm2), :] += _dot1(adj2_ref[...], xv_ref[...])
    h = o_ref[...] + b_ref[...]
    a = a_ref[0]
    zb = jnp.where(h >= 0, h, a * h)
    o_ref[...] = zb

    h1 = _dot1t(zb, hw_ref[...]) + hb_ref[...]
    hz_ref[...] = h1
    _head1_block(h1, 0, m, s_ref, c_ref, ssc_ref)


def _gcn2_layer(x_in, w, adj, b, alpha, hw, hb, bm):
    n, k = adj.shape
    dout = w.shape[0]
    hh = hw.shape[0]
    row = lambda m: (m, 0)
    const2 = lambda m: (0, 0)
    const3 = lambda m: (0, 0, 0)
    bm2 = bm // 2
    return pl.pallas_call(
        functools.partial(_gcn2_kernel, bm2=bm2),
        grid=(n // bm,),
        in_specs=[
            pl.BlockSpec((n, x_in.shape[1]), const2),
            pl.BlockSpec((dout, x_in.shape[1]), const2),
            pl.BlockSpec((bm2, k), lambda m: (2 * m, 0)),
            pl.BlockSpec((bm2, k), lambda m: (2 * m + 1, 0)),
            pl.BlockSpec((1, dout), const2),
            pl.BlockSpec(memory_space=pltpu.SMEM),
            pl.BlockSpec((hh, dout), const2),
            pl.BlockSpec((1, hh), const2),
        ],
        out_specs=[
            pl.BlockSpec((bm, dout), row),
            pl.BlockSpec((bm, hh), row),
            pl.BlockSpec((1, 1, hh), const3),
            pl.BlockSpec((1, 1, hh), const3),
            pl.BlockSpec((1, 1, hh), const3),
        ],
        out_shape=[
            jax.ShapeDtypeStruct((n, dout), jnp.float32),
            jax.ShapeDtypeStruct((n, hh), jnp.float32),
            jax.ShapeDtypeStruct((1, 1, hh), jnp.float32),
            jax.ShapeDtypeStruct((1, 1, hh), jnp.float32),
            jax.ShapeDtypeStruct((1, 1, hh), jnp.float32),
        ],
        scratch_shapes=[pltpu.VMEM((n, dout), jnp.float32)],
        compiler_params=pltpu.CompilerParams(
            dimension_semantics=("arbitrary",),
        ),
    )(x_in, w, adj, adj, b.reshape(1, dout), alpha.reshape(1),
      hw, hb.reshape(1, hh))


def _bn_of(s, c, ssc, g, n):
    mean = s / n
    var = ssc / n - (mean - c) * (mean - c)
    scale = g / jnp.sqrt(var + 1e-5)
    return mean, scale


def _bn_lin_stats_kernel(hz_ref, he_ref, hn_ref, sz_ref, cz_ref, sscz_ref,
                         se_ref, ce_ref, ssce_ref,
                         gz_ref, bez_ref, wz_ref, bz_ref,
                         ge_ref, bee_ref, we_ref, be_ref2,
                         gn_ref, ben_ref, wn_ref, bn_ref,
                         oz_ref, oe_ref, on_ref, s2_ref, c2_ref, ssc2_ref, *, n):
    m = pl.program_id(0)
    heads = ((hz_ref, oz_ref, sz_ref[0], cz_ref[0], sscz_ref[0],
              gz_ref, bez_ref, wz_ref, bz_ref),
             (he_ref, oe_ref, se_ref[0], ce_ref[0], ssce_ref[0],
              ge_ref, bee_ref, we_ref, be_ref2),
             (hn_ref, on_ref, se_ref[1], ce_ref[1], ssce_ref[1],
              gn_ref, ben_ref, wn_ref, bn_ref))
    for i, (h_ref, o_ref, s_i, c_i, ssc_i, g_ref, bei_ref, w_ref, b_ref) in \
            enumerate(heads):
        mean, scale = _bn_of(s_i, c_i, ssc_i, g_ref[...], n)
        xh = (h_ref[...] - mean) * scale + bei_ref[...]
        xh = jnp.maximum(xh, 0.0)
        h2 = _dot1t(xh, w_ref[...]) + b_ref[...]
        o_ref[...] = h2

        @pl.when(m == 0)
        def _(h2=h2, i=i):
            c2_ref[i] = jnp.mean(h2, axis=0, keepdims=True)
            s2_ref[i] = jnp.zeros_like(s2_ref[i])
            ssc2_ref[i] = jnp.zeros_like(ssc2_ref[i])

        d = h2 - c2_ref[i]
        s2_ref[i] += jnp.sum(h2, axis=0, keepdims=True)
        ssc2_ref[i] += jnp.sum(d * d, axis=0, keepdims=True)


def _bn_apply_prompt_kernel(hz_ref, he_ref, hn_ref, s_ref, c_ref, ssc_ref,
                            gz_ref, bez_ref, ge_ref, bee_ref, gn_ref, ben_ref,
                            np_ref, ap_ref, fcn_ref, fca_ref, pr_ref,
                            pab_ref, pg_ref,
                            oz_ref, oe_ref, on_ref,
                            npr_ref, apr_ref, en_ref, ea_ref, *, n):
    for i, (h_ref, o_ref, g_ref, bei_ref) in enumerate(
            ((hz_ref, oz_ref, gz_ref, bez_ref),
             (he_ref, oe_ref, ge_ref, bee_ref),
             (hn_ref, on_ref, gn_ref, ben_ref))):
        mean, scale = _bn_of(s_ref[i], c_ref[i], ssc_ref[i], g_ref[...], n)
        o_ref[...] = (h_ref[...] - mean) * scale + bei_ref[...]

    @pl.when(pl.program_id(0) == 0)
    def _():
        npr = jnp.maximum(_dot1t(np_ref[...], fcn_ref[...]), 0.0)
        apr = jnp.maximum(_dot1t(ap_ref[...], fca_ref[...]), 0.0)
        pab = pab_ref[...]
        pg = pg_ref[...]
        npr_ref[...] = npr
        apr_ref[...] = apr
        en_ref[...] = npr + jnp.maximum(_dot1t(npr, pr_ref[...]) + pab, 0.0) + pg
        ea_ref[...] = apr + jnp.maximum(_dot1t(apr, pr_ref[...]) + pab, 0.0) + pg


def _heads_and_prompts(h1z, h1e, h1n, stats_z, stats_en, prompts_in, p, bm):
    n, h = h1z.shape
    out = p['nc_W2'].shape[0]
    nm = n // bm
    row = lambda m: (m, 0)
    const2 = lambda m: (0, 0)
    const3 = lambda m: (0, 0, 0)
    arb = pltpu.CompilerParams(dimension_semantics=("arbitrary",))

    def rowspec(d):
        return pl.BlockSpec((bm, d), row)

    def statspec(d, t=3):
        return pl.BlockSpec((t, 1, d), const3)

    def statshape(d):
        return jax.ShapeDtypeStruct((3, 1, d), jnp.float32)

    def wspec(a, b):
        return pl.BlockSpec((3, a, b), const3)

    sz, cz, sscz = stats_z
    sen, cen, sscen = stats_en
    vspec = lambda d: pl.BlockSpec((1, d), const2)
    wspec2 = lambda a, b: pl.BlockSpec((a, b), const2)
    hp = [(p[k + '_g1'].reshape(1, h), p[k + '_be1'].reshape(1, h),
           p[k + '_W2'], p[k + '_b2'].reshape(1, out))
          for k in ('nc', 'ego', 'nbr')]
    hp_specs = []
    for _ in range(3):
        hp_specs += [vspec(h), vspec(h), wspec2(out, h), vspec(out)]
    hp_args = [a for t4 in hp for a in t4]
    h2z, h2e, h2n, s2, c2, ssc2 = pl.pallas_call(
        functools.partial(_bn_lin_stats_kernel, n=n),
        grid=(nm,),
        in_specs=[rowspec(h), rowspec(h), rowspec(h),
                  statspec(h, 1), statspec(h, 1), statspec(h, 1),
                  statspec(h, 2), statspec(h, 2), statspec(h, 2)] + hp_specs,
        out_specs=[rowspec(out), rowspec(out), rowspec(out),
                   statspec(out), statspec(out), statspec(out)],
        out_shape=[jax.ShapeDtypeStruct((n, out), jnp.float32)] * 3 +
                  [statshape(out)] * 3,
        compiler_params=arb,
    )(h1z, h1e, h1n, sz, cz, sscz, sen, cen, sscen, *hp_args)

    np_, ap_, fcn_w, fca_w, pr_w, pab, pg = prompts_in
    d2 = np_.shape[1]
    psh = jax.ShapeDtypeStruct((1, d2), jnp.float32)
    pspec = pl.BlockSpec((1, d2), const2)
    pwspec = pl.BlockSpec((d2, d2), const2)
    gb = []
    gb_specs = []
    for k in ('nc', 'ego', 'nbr'):
        gb += [p[k + '_g2'].reshape(1, out), p[k + '_be2'].reshape(1, out)]
        gb_specs += [vspec(out), vspec(out)]
    z, oe, on, npr, apr, en, ea = pl.pallas_call(
        functools.partial(_bn_apply_prompt_kernel, n=n),
        grid=(nm,),
        in_specs=[rowspec(out), rowspec(out), rowspec(out),
                  statspec(out), statspec(out), statspec(out)] + gb_specs +
                 [pspec, pspec, pwspec, pwspec, pwspec, pspec, pspec],
        out_specs=[rowspec(out), rowspec(out), rowspec(out),
                   pspec, pspec, pspec, pspec],
        out_shape=[jax.ShapeDtypeStruct((n, out), jnp.float32)] * 3 +
                  [psh] * 4,
        compiler_params=arb,
    )(h2z, h2e, h2n, s2, c2, ssc2, *gb,
      np_, ap_, fcn_w, fca_w, pr_w, pab, pg)
    return z, oe, on, npr, apr, en, ea


def kernel(feat, adj, ego_raw, nbr_raw, normal_prompt, abnormal_prompt, params):
    p = params
    n = adj.shape[0]
    bm = 400 if n % 400 == 0 else n
    bmh = 1000 if n % 1000 == 0 else n

    emb, h1e, h1n, s_en, c_en, ssc_en = _gcn1_layer(
        feat, p['gcn1_W'], adj, p['gcn1_b'], p['gcn1_a'],
        ego_raw, nbr_raw, p['ego_W1'], p['nbr_W1'], p['ego_b1'], p['nbr_b1'], bm)

    z_pre, h1z, s_z, c_z, ssc_z = _gcn2_layer(
        emb, p['gcn2_W'], adj, p['gcn2_b'], p['gcn2_a'],
        p['nc_W1'], p['nc_b1'], bm)
    del z_pre

    prompts_in = (normal_prompt, abnormal_prompt,
                  p['fcn_W'], p['fca_W'], p['pr_aW'],
                  p['pr_ab'].reshape(1, -1), p['pr_glob'])
    z, h_ego, h_nbr, npr, apr, en, ea = _heads_and_prompts(
        h1z, h1e, h1n, (s_z, c_z, ssc_z), (s_en, c_en, ssc_en),
        prompts_in, p, bmh)

    return (h_ego, h_nbr, npr, apr, en, ea, z)


# speedup vs baseline: 1.0354x; 1.0354x over previous
import functools

import jax
import jax.numpy as jnp
from jax.experimental import pallas as pl
from jax.experimental.pallas import tpu as pltpu


def _dot1(a, b):
    return jnp.dot(a.astype(jnp.bfloat16), b.astype(jnp.bfloat16),
                   preferred_element_type=jnp.float32)


def _dot1t(a, w):
    return jax.lax.dot_general(
        a.astype(jnp.bfloat16), w.astype(jnp.bfloat16),
        (((1,), (1,)), ((), ())), preferred_element_type=jnp.float32)


def _head1_block(h, i, m, s_ref, c_ref, ssc_ref):
    @pl.when(m == 0)
    def _():
        c_ref[i] = jnp.mean(h, axis=0, keepdims=True)
        s_ref[i] = jnp.zeros_like(s_ref[i])
        ssc_ref[i] = jnp.zeros_like(ssc_ref[i])

    d = h - c_ref[i]
    s_ref[i] += jnp.sum(h, axis=0, keepdims=True)
    ssc_ref[i] += jnp.sum(d * d, axis=0, keepdims=True)


def _gcn1_kernel(x_in_ref, w_ref, adj_ref, b_ref, a_ref, xe_ref, xn_ref,
                 hwe_ref, hwn_ref, hbe_ref, hbn_ref,
                 o_ref, he_ref, hn_ref, s_ref, c_ref, ssc_ref, xv_ref):
    m = pl.program_id(0)

    @pl.when(m == 0)
    def _():
        xv_ref[...] = _dot1t(x_in_ref[...], w_ref[...])

    o_ref[...] = jnp.zeros_like(o_ref)
    o_ref[...] += _dot1(adj_ref[...], xv_ref[...])
    h = o_ref[...] + b_ref[...]
    a = a_ref[0]
    o_ref[...] = jnp.where(h >= 0, h, a * h)

    for i, (x_ref, h_ref, hw_ref, hb_ref) in enumerate(
            ((xe_ref, he_ref, hwe_ref, hbe_ref),
             (xn_ref, hn_ref, hwn_ref, hbn_ref))):
        h1 = _dot1t(x_ref[...], hw_ref[...]) + hb_ref[...]
        h_ref[...] = h1
        _head1_block(h1, i, m, s_ref, c_ref, ssc_ref)


def _gcn1_layer(x_in, w, adj, b, alpha, ego, nbr, hwe, hwn, hbe, hbn, bm):
    n, k = adj.shape
    dout = w.shape[0]
    hh = hwe.shape[0]
    row = lambda m: (m, 0)
    const2 = lambda m: (0, 0)
    const3 = lambda m: (0, 0, 0)
    return pl.pallas_call(
        _gcn1_kernel,
        grid=(n // bm,),
        in_specs=[
            pl.BlockSpec((n, x_in.shape[1]), const2),
            pl.BlockSpec((dout, x_in.shape[1]), const2),
            pl.BlockSpec((bm, k), row),
            pl.BlockSpec((1, dout), const2),
            pl.BlockSpec(memory_space=pltpu.SMEM),
            pl.BlockSpec((bm, ego.shape[1]), row),
            pl.BlockSpec((bm, nbr.shape[1]), row),
            pl.BlockSpec((hh, ego.shape[1]), const2),
            pl.BlockSpec((hh, nbr.shape[1]), const2),
            pl.BlockSpec((1, hh), const2),
            pl.BlockSpec((1, hh), const2),
        ],
        out_specs=[
            pl.BlockSpec((bm, dout), row),
            pl.BlockSpec((bm, hh), row),
            pl.BlockSpec((bm, hh), row),
            pl.BlockSpec((2, 1, hh), const3),
            pl.BlockSpec((2, 1, hh), const3),
            pl.BlockSpec((2, 1, hh), const3),
        ],
        out_shape=[
            jax.ShapeDtypeStruct((n, dout), jnp.float32),
            jax.ShapeDtypeStruct((n, hh), jnp.float32),
            jax.ShapeDtypeStruct((n, hh), jnp.float32),
            jax.ShapeDtypeStruct((2, 1, hh), jnp.float32),
            jax.ShapeDtypeStruct((2, 1, hh), jnp.float32),
            jax.ShapeDtypeStruct((2, 1, hh), jnp.float32),
        ],
        scratch_shapes=[pltpu.VMEM((n, dout), jnp.float32)],
        compiler_params=pltpu.CompilerParams(
            dimension_semantics=("arbitrary",),
        ),
    )(x_in, w, adj, b.reshape(1, dout), alpha.reshape(1), ego, nbr,
      hwe, hwn, hbe.reshape(1, hh), hbn.reshape(1, hh))


def _gcn2_kernel(x_in_ref, w_ref, adj_ref, b_ref, a_ref, hw_ref, hb_ref,
                 o_ref, hz_ref, s_ref, c_ref, ssc_ref, xv_ref):
    m = pl.program_id(0)

    @pl.when(m == 0)
    def _():
        xv_ref[...] = _dot1t(x_in_ref[...], w_ref[...])

    o_ref[...] = jnp.zeros_like(o_ref)
    o_ref[...] += _dot1(adj_ref[...], xv_ref[...])
    h = o_ref[...] + b_ref[...]
    a = a_ref[0]
    zb = jnp.where(h >= 0, h, a * h)
    o_ref[...] = zb

    h1 = _dot1t(zb, hw_ref[...]) + hb_ref[...]
    hz_ref[...] = h1
    _head1_block(h1, 0, m, s_ref, c_ref, ssc_ref)


def _gcn2_layer(x_in, w, adj, b, alpha, hw, hb, bm):
    n, k = adj.shape
    dout = w.shape[0]
    hh = hw.shape[0]
    row = lambda m: (m, 0)
    const2 = lambda m: (0, 0)
    const3 = lambda m: (0, 0, 0)
    return pl.pallas_call(
        _gcn2_kernel,
        grid=(n // bm,),
        in_specs=[
            pl.BlockSpec((n, x_in.shape[1]), const2),
            pl.BlockSpec((dout, x_in.shape[1]), const2),
            pl.BlockSpec((bm, k), row),
            pl.BlockSpec((1, dout), const2),
            pl.BlockSpec(memory_space=pltpu.SMEM),
            pl.BlockSpec((hh, dout), const2),
            pl.BlockSpec((1, hh), const2),
        ],
        out_specs=[
            pl.BlockSpec((bm, dout), row),
            pl.BlockSpec((bm, hh), row),
            pl.BlockSpec((1, 1, hh), const3),
            pl.BlockSpec((1, 1, hh), const3),
            pl.BlockSpec((1, 1, hh), const3),
        ],
        out_shape=[
            jax.ShapeDtypeStruct((n, dout), jnp.float32),
            jax.ShapeDtypeStruct((n, hh), jnp.float32),
            jax.ShapeDtypeStruct((1, 1, hh), jnp.float32),
            jax.ShapeDtypeStruct((1, 1, hh), jnp.float32),
            jax.ShapeDtypeStruct((1, 1, hh), jnp.float32),
        ],
        scratch_shapes=[pltpu.VMEM((n, dout), jnp.float32)],
        compiler_params=pltpu.CompilerParams(
            dimension_semantics=("arbitrary",),
        ),
    )(x_in, w, adj, b.reshape(1, dout), alpha.reshape(1),
      hw, hb.reshape(1, hh))


def _bn_of(s, c, ssc, g, n):
    mean = s / n
    var = ssc / n - (mean - c) * (mean - c)
    scale = g / jnp.sqrt(var + 1e-5)
    return mean, scale


def _bn_lin_stats_kernel(hz_ref, he_ref, hn_ref, sz_ref, cz_ref, sscz_ref,
                         se_ref, ce_ref, ssce_ref,
                         gz_ref, bez_ref, wz_ref, bz_ref,
                         ge_ref, bee_ref, we_ref, be_ref2,
                         gn_ref, ben_ref, wn_ref, bn_ref,
                         oz_ref, oe_ref, on_ref, s2_ref, c2_ref, ssc2_ref, *, n):
    m = pl.program_id(0)
    heads = ((hz_ref, oz_ref, sz_ref[0], cz_ref[0], sscz_ref[0],
              gz_ref, bez_ref, wz_ref, bz_ref),
             (he_ref, oe_ref, se_ref[0], ce_ref[0], ssce_ref[0],
              ge_ref, bee_ref, we_ref, be_ref2),
             (hn_ref, on_ref, se_ref[1], ce_ref[1], ssce_ref[1],
              gn_ref, ben_ref, wn_ref, bn_ref))
    for i, (h_ref, o_ref, s_i, c_i, ssc_i, g_ref, bei_ref, w_ref, b_ref) in \
            enumerate(heads):
        mean, scale = _bn_of(s_i, c_i, ssc_i, g_ref[...], n)
        xh = (h_ref[...] - mean) * scale + bei_ref[...]
        xh = jnp.maximum(xh, 0.0)
        h2 = _dot1t(xh, w_ref[...]) + b_ref[...]
        o_ref[...] = h2

        @pl.when(m == 0)
        def _(h2=h2, i=i):
            c2_ref[i] = jnp.mean(h2, axis=0, keepdims=True)
            s2_ref[i] = jnp.zeros_like(s2_ref[i])
            ssc2_ref[i] = jnp.zeros_like(ssc2_ref[i])

        d = h2 - c2_ref[i]
        s2_ref[i] += jnp.sum(h2, axis=0, keepdims=True)
        ssc2_ref[i] += jnp.sum(d * d, axis=0, keepdims=True)


def _bn_apply_prompt_kernel(hz_ref, he_ref, hn_ref, s_ref, c_ref, ssc_ref,
                            gz_ref, bez_ref, ge_ref, bee_ref, gn_ref, ben_ref,
                            np_ref, ap_ref, fcn_ref, fca_ref, pr_ref,
                            pab_ref, pg_ref,
                            oz_ref, oe_ref, on_ref,
                            npr_ref, apr_ref, en_ref, ea_ref, *, n):
    for i, (h_ref, o_ref, g_ref, bei_ref) in enumerate(
            ((hz_ref, oz_ref, gz_ref, bez_ref),
             (he_ref, oe_ref, ge_ref, bee_ref),
             (hn_ref, on_ref, gn_ref, ben_ref))):
        mean, scale = _bn_of(s_ref[i], c_ref[i], ssc_ref[i], g_ref[...], n)
        o_ref[...] = (h_ref[...] - mean) * scale + bei_ref[...]

    @pl.when(pl.program_id(0) == 0)
    def _():
        npr = jnp.maximum(_dot1t(np_ref[...], fcn_ref[...]), 0.0)
        apr = jnp.maximum(_dot1t(ap_ref[...], fca_ref[...]), 0.0)
        pab = pab_ref[...]
        pg = pg_ref[...]
        npr_ref[...] = npr
        apr_ref[...] = apr
        en_ref[...] = npr + jnp.maximum(_dot1t(npr, pr_ref[...]) + pab, 0.0) + pg
        ea_ref[...] = apr + jnp.maximum(_dot1t(apr, pr_ref[...]) + pab, 0.0) + pg


def _heads_and_prompts(h1z, h1e, h1n, stats_z, stats_en, prompts_in, p, bm):
    n, h = h1z.shape
    out = p['nc_W2'].shape[0]
    nm = n // bm
    row = lambda m: (m, 0)
    const2 = lambda m: (0, 0)
    const3 = lambda m: (0, 0, 0)
    arb = pltpu.CompilerParams(dimension_semantics=("arbitrary",))

    def rowspec(d):
        return pl.BlockSpec((bm, d), row)

    def statspec(d, t=3):
        return pl.BlockSpec((t, 1, d), const3)

    def statshape(d):
        return jax.ShapeDtypeStruct((3, 1, d), jnp.float32)

    def wspec(a, b):
        return pl.BlockSpec((3, a, b), const3)

    sz, cz, sscz = stats_z
    sen, cen, sscen = stats_en
    vspec = lambda d: pl.BlockSpec((1, d), const2)
    wspec2 = lambda a, b: pl.BlockSpec((a, b), const2)
    hp = [(p[k + '_g1'].reshape(1, h), p[k + '_be1'].reshape(1, h),
           p[k + '_W2'], p[k + '_b2'].reshape(1, out))
          for k in ('nc', 'ego', 'nbr')]
    hp_specs = []
    for _ in range(3):
        hp_specs += [vspec(h), vspec(h), wspec2(out, h), vspec(out)]
    hp_args = [a for t4 in hp for a in t4]
    h2z, h2e, h2n, s2, c2, ssc2 = pl.pallas_call(
        functools.partial(_bn_lin_stats_kernel, n=n),
        grid=(nm,),
        in_specs=[rowspec(h), rowspec(h), rowspec(h),
                  statspec(h, 1), statspec(h, 1), statspec(h, 1),
                  statspec(h, 2), statspec(h, 2), statspec(h, 2)] + hp_specs,
        out_specs=[rowspec(out), rowspec(out), rowspec(out),
                   statspec(out), statspec(out), statspec(out)],
        out_shape=[jax.ShapeDtypeStruct((n, out), jnp.float32)] * 3 +
                  [statshape(out)] * 3,
        compiler_params=arb,
    )(h1z, h1e, h1n, sz, cz, sscz, sen, cen, sscen, *hp_args)

    np_, ap_, fcn_w, fca_w, pr_w, pab, pg = prompts_in
    d2 = np_.shape[1]
    psh = jax.ShapeDtypeStruct((1, d2), jnp.float32)
    pspec = pl.BlockSpec((1, d2), const2)
    pwspec = pl.BlockSpec((d2, d2), const2)
    gb = []
    gb_specs = []
    for k in ('nc', 'ego', 'nbr'):
        gb += [p[k + '_g2'].reshape(1, out), p[k + '_be2'].reshape(1, out)]
        gb_specs += [vspec(out), vspec(out)]
    z, oe, on, npr, apr, en, ea = pl.pallas_call(
        functools.partial(_bn_apply_prompt_kernel, n=n),
        grid=(nm,),
        in_specs=[rowspec(out), rowspec(out), rowspec(out),
                  statspec(out), statspec(out), statspec(out)] + gb_specs +
                 [pspec, pspec, pwspec, pwspec, pwspec, pspec, pspec],
        out_specs=[rowspec(out), rowspec(out), rowspec(out),
                   pspec, pspec, pspec, pspec],
        out_shape=[jax.ShapeDtypeStruct((n, out), jnp.float32)] * 3 +
                  [psh] * 4,
        compiler_params=arb,
    )(h2z, h2e, h2n, s2, c2, ssc2, *gb,
      np_, ap_, fcn_w, fca_w, pr_w, pab, pg)
    return z, oe, on, npr, apr, en, ea


def kernel(feat, adj, ego_raw, nbr_raw, normal_prompt, abnormal_prompt, params):
    p = params
    n = adj.shape[0]
    bm = 400 if n % 400 == 0 else n
    bmh = 2000 if n % 2000 == 0 else n

    emb, h1e, h1n, s_en, c_en, ssc_en = _gcn1_layer(
        feat, p['gcn1_W'], adj, p['gcn1_b'], p['gcn1_a'],
        ego_raw, nbr_raw, p['ego_W1'], p['nbr_W1'], p['ego_b1'], p['nbr_b1'], bm)

    z_pre, h1z, s_z, c_z, ssc_z = _gcn2_layer(
        emb, p['gcn2_W'], adj, p['gcn2_b'], p['gcn2_a'],
        p['nc_W1'], p['nc_b1'], bm)
    del z_pre

    prompts_in = (normal_prompt, abnormal_prompt,
                  p['fcn_W'], p['fca_W'], p['pr_aW'],
                  p['pr_ab'].reshape(1, -1), p['pr_glob'])
    z, h_ego, h_nbr, npr, apr, en, ea = _heads_and_prompts(
        h1z, h1e, h1n, (s_z, c_z, ssc_z), (s_en, c_en, ssc_en),
        prompts_in, p, bmh)

    return (h_ego, h_nbr, npr, apr, en, ea, z)


# speedup vs baseline: 1.0392x; 1.0036x over previous
import functools

import jax
import jax.numpy as jnp
from jax.experimental import pallas as pl
from jax.experimental.pallas import tpu as pltpu


def _dot1(a, b):
    return jnp.dot(a.astype(jnp.bfloat16), b.astype(jnp.bfloat16),
                   preferred_element_type=jnp.float32)


def _dot1t(a, w):
    return jax.lax.dot_general(
        a.astype(jnp.bfloat16), w.astype(jnp.bfloat16),
        (((1,), (1,)), ((), ())), preferred_element_type=jnp.float32)


def _head1_block(h, i, m, s_ref, c_ref, ssc_ref):
    @pl.when(m == 0)
    def _():
        c_ref[i] = jnp.mean(h, axis=0, keepdims=True)
        s_ref[i] = jnp.zeros_like(s_ref[i])
        ssc_ref[i] = jnp.zeros_like(ssc_ref[i])

    d = h - c_ref[i]
    s_ref[i] += jnp.sum(h, axis=0, keepdims=True)
    ssc_ref[i] += jnp.sum(d * d, axis=0, keepdims=True)


def _gcn1_kernel(x_in_ref, w_ref, adj_ref, b_ref, a_ref, xe_ref, xn_ref,
                 hwe_ref, hwn_ref, hbe_ref, hbn_ref,
                 o_ref, he_ref, hn_ref, s_ref, c_ref, ssc_ref, xv_ref):
    m = pl.program_id(0)

    @pl.when(m == 0)
    def _():
        xv_ref[...] = _dot1t(x_in_ref[...], w_ref[...])

    o_ref[...] = jnp.zeros_like(o_ref)
    o_ref[...] += _dot1(adj_ref[...], xv_ref[...])
    h = o_ref[...] + b_ref[...]
    a = a_ref[0]
    o_ref[...] = jnp.where(h >= 0, h, a * h)

    for i, (x_ref, h_ref, hw_ref, hb_ref) in enumerate(
            ((xe_ref, he_ref, hwe_ref, hbe_ref),
             (xn_ref, hn_ref, hwn_ref, hbn_ref))):
        h1 = _dot1t(x_ref[...], hw_ref[...]) + hb_ref[...]
        h_ref[...] = h1
        _head1_block(h1, i, m, s_ref, c_ref, ssc_ref)


def _gcn1_layer(x_in, w, adj, b, alpha, ego, nbr, hwe, hwn, hbe, hbn, bm):
    n, k = adj.shape
    dout = w.shape[0]
    hh = hwe.shape[0]
    row = lambda m: (m, 0)
    const2 = lambda m: (0, 0)
    const3 = lambda m: (0, 0, 0)
    return pl.pallas_call(
        _gcn1_kernel,
        grid=(n // bm,),
        in_specs=[
            pl.BlockSpec((n, x_in.shape[1]), const2),
            pl.BlockSpec((dout, x_in.shape[1]), const2),
            pl.BlockSpec((bm, k), row),
            pl.BlockSpec((1, dout), const2),
            pl.BlockSpec(memory_space=pltpu.SMEM),
            pl.BlockSpec((bm, ego.shape[1]), row),
            pl.BlockSpec((bm, nbr.shape[1]), row),
            pl.BlockSpec((hh, ego.shape[1]), const2),
            pl.BlockSpec((hh, nbr.shape[1]), const2),
            pl.BlockSpec((1, hh), const2),
            pl.BlockSpec((1, hh), const2),
        ],
        out_specs=[
            pl.BlockSpec((bm, dout), row),
            pl.BlockSpec((bm, hh), row),
            pl.BlockSpec((bm, hh), row),
            pl.BlockSpec((2, 1, hh), const3),
            pl.BlockSpec((2, 1, hh), const3),
            pl.BlockSpec((2, 1, hh), const3),
        ],
        out_shape=[
            jax.ShapeDtypeStruct((n, dout), jnp.float32),
            jax.ShapeDtypeStruct((n, hh), jnp.float32),
            jax.ShapeDtypeStruct((n, hh), jnp.float32),
            jax.ShapeDtypeStruct((2, 1, hh), jnp.float32),
            jax.ShapeDtypeStruct((2, 1, hh), jnp.float32),
            jax.ShapeDtypeStruct((2, 1, hh), jnp.float32),
        ],
        scratch_shapes=[pltpu.VMEM((n, dout), jnp.float32)],
        compiler_params=pltpu.CompilerParams(
            dimension_semantics=("arbitrary",),
        ),
    )(x_in, w, adj, b.reshape(1, dout), alpha.reshape(1), ego, nbr,
      hwe, hwn, hbe.reshape(1, hh), hbn.reshape(1, hh))


def _gcn2_kernel(x_in_ref, w_ref, adj_ref, b_ref, a_ref, hw_ref, hb_ref,
                 hz_ref, s_ref, c_ref, ssc_ref, xv_ref, acc_ref):
    m = pl.program_id(0)

    @pl.when(m == 0)
    def _():
        xv_ref[...] = _dot1t(x_in_ref[...], w_ref[...])

    acc_ref[...] = jnp.zeros_like(acc_ref)
    acc_ref[...] += _dot1(adj_ref[...], xv_ref[...])
    h = acc_ref[...] + b_ref[...]
    a = a_ref[0]
    zb = jnp.where(h >= 0, h, a * h)

    h1 = _dot1t(zb, hw_ref[...]) + hb_ref[...]
    hz_ref[...] = h1
    _head1_block(h1, 0, m, s_ref, c_ref, ssc_ref)


def _gcn2_layer(x_in, w, adj, b, alpha, hw, hb, bm):
    n, k = adj.shape
    dout = w.shape[0]
    hh = hw.shape[0]
    row = lambda m: (m, 0)
    const2 = lambda m: (0, 0)
    const3 = lambda m: (0, 0, 0)
    return pl.pallas_call(
        _gcn2_kernel,
        grid=(n // bm,),
        in_specs=[
            pl.BlockSpec((n, x_in.shape[1]), const2),
            pl.BlockSpec((dout, x_in.shape[1]), const2),
            pl.BlockSpec((bm, k), row),
            pl.BlockSpec((1, dout), const2),
            pl.BlockSpec(memory_space=pltpu.SMEM),
            pl.BlockSpec((hh, dout), const2),
            pl.BlockSpec((1, hh), const2),
        ],
        out_specs=[
            pl.BlockSpec((bm, hh), row),
            pl.BlockSpec((1, 1, hh), const3),
            pl.BlockSpec((1, 1, hh), const3),
            pl.BlockSpec((1, 1, hh), const3),
        ],
        out_shape=[
            jax.ShapeDtypeStruct((n, hh), jnp.float32),
            jax.ShapeDtypeStruct((1, 1, hh), jnp.float32),
            jax.ShapeDtypeStruct((1, 1, hh), jnp.float32),
            jax.ShapeDtypeStruct((1, 1, hh), jnp.float32),
        ],
        scratch_shapes=[pltpu.VMEM((n, dout), jnp.float32),
                        pltpu.VMEM((bm, dout), jnp.float32)],
        compiler_params=pltpu.CompilerParams(
            dimension_semantics=("arbitrary",),
        ),
    )(x_in, w, adj, b.reshape(1, dout), alpha.reshape(1),
      hw, hb.reshape(1, hh))


def _bn_of(s, c, ssc, g, n):
    mean = s / n
    var = ssc / n - (mean - c) * (mean - c)
    scale = g / jnp.sqrt(var + 1e-5)
    return mean, scale


def _bn_lin_stats_kernel(hz_ref, he_ref, hn_ref, sz_ref, cz_ref, sscz_ref,
                         se_ref, ce_ref, ssce_ref,
                         gz_ref, bez_ref, wz_ref, bz_ref,
                         ge_ref, bee_ref, we_ref, be_ref2,
                         gn_ref, ben_ref, wn_ref, bn_ref,
                         oz_ref, oe_ref, on_ref, s2_ref, c2_ref, ssc2_ref, *, n):
    m = pl.program_id(0)
    heads = ((hz_ref, oz_ref, sz_ref[0], cz_ref[0], sscz_ref[0],
              gz_ref, bez_ref, wz_ref, bz_ref),
             (he_ref, oe_ref, se_ref[0], ce_ref[0], ssce_ref[0],
              ge_ref, bee_ref, we_ref, be_ref2),
             (hn_ref, on_ref, se_ref[1], ce_ref[1], ssce_ref[1],
              gn_ref, ben_ref, wn_ref, bn_ref))
    for i, (h_ref, o_ref, s_i, c_i, ssc_i, g_ref, bei_ref, w_ref, b_ref) in \
            enumerate(heads):
        mean, scale = _bn_of(s_i, c_i, ssc_i, g_ref[...], n)
        xh = (h_ref[...] - mean) * scale + bei_ref[...]
        xh = jnp.maximum(xh, 0.0)
        h2 = _dot1t(xh, w_ref[...]) + b_ref[...]
        o_ref[...] = h2

        @pl.when(m == 0)
        def _(h2=h2, i=i):
            c2_ref[i] = jnp.mean(h2, axis=0, keepdims=True)
            s2_ref[i] = jnp.zeros_like(s2_ref[i])
            ssc2_ref[i] = jnp.zeros_like(ssc2_ref[i])

        d = h2 - c2_ref[i]
        s2_ref[i] += jnp.sum(h2, axis=0, keepdims=True)
        ssc2_ref[i] += jnp.sum(d * d, axis=0, keepdims=True)


def _bn_apply_prompt_kernel(hz_ref, he_ref, hn_ref, s_ref, c_ref, ssc_ref,
                            gz_ref, bez_ref, ge_ref, bee_ref, gn_ref, ben_ref,
                            np_ref, ap_ref, fcn_ref, fca_ref, pr_ref,
                            pab_ref, pg_ref,
                            oz_ref, oe_ref, on_ref,
                            npr_ref, apr_ref, en_ref, ea_ref, *, n):
    for i, (h_ref, o_ref, g_ref, bei_ref) in enumerate(
            ((hz_ref, oz_ref, gz_ref, bez_ref),
             (he_ref, oe_ref, ge_ref, bee_ref),
             (hn_ref, on_ref, gn_ref, ben_ref))):
        mean, scale = _bn_of(s_ref[i], c_ref[i], ssc_ref[i], g_ref[...], n)
        o_ref[...] = (h_ref[...] - mean) * scale + bei_ref[...]

    @pl.when(pl.program_id(0) == 0)
    def _():
        npr = jnp.maximum(_dot1t(np_ref[...], fcn_ref[...]), 0.0)
        apr = jnp.maximum(_dot1t(ap_ref[...], fca_ref[...]), 0.0)
        pab = pab_ref[...]
        pg = pg_ref[...]
        npr_ref[...] = npr
        apr_ref[...] = apr
        en_ref[...] = npr + jnp.maximum(_dot1t(npr, pr_ref[...]) + pab, 0.0) + pg
        ea_ref[...] = apr + jnp.maximum(_dot1t(apr, pr_ref[...]) + pab, 0.0) + pg


def _heads_and_prompts(h1z, h1e, h1n, stats_z, stats_en, prompts_in, p, bm):
    n, h = h1z.shape
    out = p['nc_W2'].shape[0]
    nm = n // bm
    row = lambda m: (m, 0)
    const2 = lambda m: (0, 0)
    const3 = lambda m: (0, 0, 0)
    arb = pltpu.CompilerParams(dimension_semantics=("arbitrary",))

    def rowspec(d):
        return pl.BlockSpec((bm, d), row)

    def statspec(d, t=3):
        return pl.BlockSpec((t, 1, d), const3)

    def statshape(d):
        return jax.ShapeDtypeStruct((3, 1, d), jnp.float32)

    def wspec(a, b):
        return pl.BlockSpec((3, a, b), const3)

    sz, cz, sscz = stats_z
    sen, cen, sscen = stats_en
    vspec = lambda d: pl.BlockSpec((1, d), const2)
    wspec2 = lambda a, b: pl.BlockSpec((a, b), const2)
    hp = [(p[k + '_g1'].reshape(1, h), p[k + '_be1'].reshape(1, h),
           p[k + '_W2'], p[k + '_b2'].reshape(1, out))
          for k in ('nc', 'ego', 'nbr')]
    hp_specs = []
    for _ in range(3):
        hp_specs += [vspec(h), vspec(h), wspec2(out, h), vspec(out)]
    hp_args = [a for t4 in hp for a in t4]
    h2z, h2e, h2n, s2, c2, ssc2 = pl.pallas_call(
        functools.partial(_bn_lin_stats_kernel, n=n),
        grid=(nm,),
        in_specs=[rowspec(h), rowspec(h), rowspec(h),
                  statspec(h, 1), statspec(h, 1), statspec(h, 1),
                  statspec(h, 2), statspec(h, 2), statspec(h, 2)] + hp_specs,
        out_specs=[rowspec(out), rowspec(out), rowspec(out),
                   statspec(out), statspec(out), statspec(out)],
        out_shape=[jax.ShapeDtypeStruct((n, out), jnp.float32)] * 3 +
                  [statshape(out)] * 3,
        compiler_params=arb,
    )(h1z, h1e, h1n, sz, cz, sscz, sen, cen, sscen, *hp_args)

    np_, ap_, fcn_w, fca_w, pr_w, pab, pg = prompts_in
    d2 = np_.shape[1]
    psh = jax.ShapeDtypeStruct((1, d2), jnp.float32)
    pspec = pl.BlockSpec((1, d2), const2)
    pwspec = pl.BlockSpec((d2, d2), const2)
    gb = []
    gb_specs = []
    for k in ('nc', 'ego', 'nbr'):
        gb += [p[k + '_g2'].reshape(1, out), p[k + '_be2'].reshape(1, out)]
        gb_specs += [vspec(out), vspec(out)]
    z, oe, on, npr, apr, en, ea = pl.pallas_call(
        functools.partial(_bn_apply_prompt_kernel, n=n),
        grid=(nm,),
        in_specs=[rowspec(out), rowspec(out), rowspec(out),
                  statspec(out), statspec(out), statspec(out)] + gb_specs +
                 [pspec, pspec, pwspec, pwspec, pwspec, pspec, pspec],
        out_specs=[rowspec(out), rowspec(out), rowspec(out),
                   pspec, pspec, pspec, pspec],
        out_shape=[jax.ShapeDtypeStruct((n, out), jnp.float32)] * 3 +
                  [psh] * 4,
        compiler_params=arb,
    )(h2z, h2e, h2n, s2, c2, ssc2, *gb,
      np_, ap_, fcn_w, fca_w, pr_w, pab, pg)
    return z, oe, on, npr, apr, en, ea


def kernel(feat, adj, ego_raw, nbr_raw, normal_prompt, abnormal_prompt, params):
    p = params
    n = adj.shape[0]
    bm = 400 if n % 400 == 0 else n
    bmh = 5000 if n % 5000 == 0 else n

    emb, h1e, h1n, s_en, c_en, ssc_en = _gcn1_layer(
        feat, p['gcn1_W'], adj, p['gcn1_b'], p['gcn1_a'],
        ego_raw, nbr_raw, p['ego_W1'], p['nbr_W1'], p['ego_b1'], p['nbr_b1'], bm)

    h1z, s_z, c_z, ssc_z = _gcn2_layer(
        emb, p['gcn2_W'], adj, p['gcn2_b'], p['gcn2_a'],
        p['nc_W1'], p['nc_b1'], bm)

    prompts_in = (normal_prompt, abnormal_prompt,
                  p['fcn_W'], p['fca_W'], p['pr_aW'],
                  p['pr_ab'].reshape(1, -1), p['pr_glob'])
    z, h_ego, h_nbr, npr, apr, en, ea = _heads_and_prompts(
        h1z, h1e, h1n, (s_z, c_z, ssc_z), (s_en, c_en, ssc_en),
        prompts_in, p, bmh)

    return (h_ego, h_nbr, npr, apr, en, ea, z)
